# baseline (speedup 1.0000x reference)
import functools

import jax
import jax.numpy as jnp
from jax import lax
from jax.experimental import pallas as pl
from jax.experimental.pallas import tpu as pltpu
from jax.experimental.pallas import tpu_sc as plsc

R = 32768
K = 64
J = 256
L = 16
NC = 2
NS = 16
NW = NC * NS
RPW = R // NW
BLK = 128
NBLK = RPW // BLK
NCH = J // L

_mesh = plsc.VectorSubcoreMesh(core_axis_name="c", subcore_axis_name="s")


@functools.partial(
    pl.kernel,
    out_type=jax.ShapeDtypeStruct((R, J), jnp.float32),
    mesh=_mesh,
    compiler_params=pltpu.CompilerParams(needs_layout_passes=False, use_tc_tiling_on_sc=False),
    scratch_types=[
        pltpu.VMEM((J,), jnp.int32),
        pltpu.VMEM((J,), jnp.float32),
        pltpu.VMEM((J,), jnp.float32),
        pltpu.VMEM((L,), jnp.float32),
        pltpu.VMEM((J,), jnp.int32),
        pltpu.VMEM((J,), jnp.float32),
        pltpu.VMEM((J,), jnp.float32),
        pltpu.VMEM((J,), jnp.float32),
        pltpu.VMEM((RPW * K,), jnp.float32),
        pltpu.VMEM((RPW, L), jnp.float32),
        pltpu.VMEM((RPW, L), jnp.float32),
        pltpu.SemaphoreType.DMA,
        pltpu.SemaphoreType.DMA,
    ],
)
def _regrid(y_hbm, inds_hbm, w_hbm, mask_hbm, fill_hbm, out_hbm,
            s_inds, s_w, s_mask, s_fill, s_i0, s_a0, s_a1, s_c,
            y_all, o_v0, o_v1, osem0, osem1):
    wid = lax.axis_index("s") * NC + lax.axis_index("c")
    row0 = wid * RPW

    pltpu.sync_copy(inds_hbm, s_inds)
    pltpu.sync_copy(w_hbm, s_w)
    pltpu.sync_copy(mask_hbm, s_mask)
    pltpu.sync_copy(fill_hbm, s_fill)

    fillv = s_fill[...]
    for c in range(NCH):
        sl = pl.ds(c * L, L)
        iv = s_inds[sl]
        wv = s_w[sl]
        mv = s_mask[sl]
        a1 = wv * mv
        s_i0[sl] = iv - 1
        s_a0[sl] = mv - a1
        s_a1[sl] = a1
        s_c[sl] = fillv - fillv * mv

    pltpu.sync_copy(y_hbm.at[pl.ds(row0 * K, RPW * K)], y_all)

    osems = (osem0, osem1)
    o_bufs = (o_v0, o_v1)
    out_d = [None, None]
    for c in range(NCH):
        p = c % 2
        sl = pl.ds(c * L, L)
        i0 = s_i0[sl]
        a0 = s_a0[sl]
        a1 = s_a1[sl]
        cv = s_c[sl]
        if out_d[p] is not None:
            out_d[p].wait()
        ob = o_bufs[p]

        @plsc.parallel_loop(0, RPW, unroll=8)
        def row_loop(r, ob=ob, i0=i0, a0=a0, a1=a1, cv=cv):
            bidx = i0 + r * K
            y0 = plsc.load_gather(y_all, [bidx])
            y1 = plsc.load_gather(y_all, [bidx + 1])
            ob[r, :] = cv + a0 * y0 + a1 * y1

        out_d[p] = pltpu.async_copy(
            ob, out_hbm.at[pl.ds(row0, RPW), pl.ds(c * L, L)], osems[p])
    out_d[0].wait()
    out_d[1].wait()


def kernel(y_data, inds, weights, mask, fill_value):
    y_flat = y_data.reshape(-1)
    inds32 = inds.astype(jnp.int32)
    w = weights.astype(jnp.float32)
    m = mask.astype(jnp.float32)
    fill16 = jnp.broadcast_to(jnp.asarray(fill_value, jnp.float32), (L,))
    return _regrid(y_flat, inds32, w, m, fill16)


# speedup vs baseline: 1.0238x; 1.0238x over previous
import functools

import jax
import jax.numpy as jnp
from jax import lax
from jax.experimental import pallas as pl
from jax.experimental.pallas import tpu as pltpu
from jax.experimental.pallas import tpu_sc as plsc

R = 32768
K = 64
J = 256
L = 16
NC = 2
NS = 16
NW = NC * NS
RPW = R // NW
BLK = 128
NBLK = RPW // BLK
NCH = J // L

_mesh = plsc.VectorSubcoreMesh(core_axis_name="c", subcore_axis_name="s")


@functools.partial(
    pl.kernel,
    out_type=jax.ShapeDtypeStruct((R, J), jnp.float32),
    mesh=_mesh,
    compiler_params=pltpu.CompilerParams(needs_layout_passes=False, use_tc_tiling_on_sc=False),
    scratch_types=[
        pltpu.VMEM((J,), jnp.int32),
        pltpu.VMEM((J,), jnp.float32),
        pltpu.VMEM((J,), jnp.float32),
        pltpu.VMEM((L,), jnp.float32),
        pltpu.VMEM((J,), jnp.int32),
        pltpu.VMEM((J,), jnp.float32),
        pltpu.VMEM((J,), jnp.float32),
        pltpu.VMEM((J,), jnp.float32),
        pltpu.VMEM((BLK * K,), jnp.float32),
        pltpu.VMEM((BLK * K,), jnp.float32),
        pltpu.VMEM((BLK, J), jnp.float32),
        pltpu.VMEM((BLK, J), jnp.float32),
        pltpu.SemaphoreType.DMA,
        pltpu.SemaphoreType.DMA,
        pltpu.SemaphoreType.DMA,
        pltpu.SemaphoreType.DMA,
    ],
)
def _regrid(y_hbm, inds_hbm, w_hbm, mask_hbm, fill_hbm, out_hbm,
            s_inds, s_w, s_mask, s_fill, s_i0, s_a0, s_a1, s_c,
            y_v0, y_v1, o_v0, o_v1, isem0, isem1, osem0, osem1):
    wid = lax.axis_index("s") * NC + lax.axis_index("c")
    row0 = wid * RPW

    pltpu.sync_copy(inds_hbm, s_inds)
    pltpu.sync_copy(w_hbm, s_w)
    pltpu.sync_copy(mask_hbm, s_mask)
    pltpu.sync_copy(fill_hbm, s_fill)

    fillv = s_fill[...]
    for c in range(NCH):
        sl = pl.ds(c * L, L)
        iv = s_inds[sl]
        wv = s_w[sl]
        mv = s_mask[sl]
        a1 = wv * mv
        s_i0[sl] = iv - 1
        s_a0[sl] = mv - a1
        s_a1[sl] = a1
        s_c[sl] = fillv - fillv * mv

    isems = (isem0, isem1)
    osems = (osem0, osem1)
    y_bufs = (y_v0, y_v1)
    o_bufs = (o_v0, o_v1)

    def start_in(b):
        base = (row0 + b * BLK) * K
        return pltpu.async_copy(
            y_hbm.at[pl.ds(base, BLK * K)], y_bufs[b % 2], isems[b % 2])

    def start_out(b):
        return pltpu.async_copy(
            o_bufs[b % 2], out_hbm.at[pl.ds(row0 + b * BLK, BLK), :],
            osems[b % 2])

    in_d = start_in(0)
    out_d = [None, None]
    for b in range(NBLK):
        p = b % 2
        nxt = start_in(b + 1) if b + 1 < NBLK else None
        in_d.wait()
        if out_d[p] is not None:
            out_d[p].wait()
        yvp = y_bufs[p]
        ovp = o_bufs[p]

        def chunk_body(c, _, yvp=yvp, ovp=ovp):
            sl = pl.ds(c * L, L)
            i0 = s_i0[sl]
            a0 = s_a0[sl]
            a1 = s_a1[sl]
            cv = s_c[sl]

            @plsc.parallel_loop(0, BLK, unroll=8)
            def row_loop(r):
                bidx = i0 + r * K
                y0 = plsc.load_gather(yvp, [bidx])
                y1 = plsc.load_gather(yvp, [bidx + 1])
                ovp[r, sl] = cv + a0 * y0 + a1 * y1

            return 0

        lax.fori_loop(0, NCH, chunk_body, 0)
        out_d[p] = start_out(b)
        in_d = nxt
    out_d[0].wait()
    out_d[1].wait()


def kernel(y_data, inds, weights, mask, fill_value):
    y_flat = y_data.reshape(-1)
    inds32 = inds.astype(jnp.int32)
    w = weights.astype(jnp.float32)
    m = mask.astype(jnp.float32)
    fill16 = jnp.broadcast_to(jnp.asarray(fill_value, jnp.float32), (L,))
    return _regrid(y_flat, inds32, w, m, fill16)


# speedup vs baseline: 1.5376x; 1.5018x over previous
import functools

import jax
import jax.numpy as jnp
from jax import lax
from jax.experimental import pallas as pl
from jax.experimental.pallas import tpu as pltpu
from jax.experimental.pallas import tpu_sc as plsc

R = 32768
K = 64
J = 256
L = 16
NC = 2
NS = 16
NW = NC * NS
RPW = R // NW
BLK = 128
NBLK = RPW // BLK
NCH = J // L

_mesh = plsc.VectorSubcoreMesh(core_axis_name="c", subcore_axis_name="s")


@functools.partial(
    pl.kernel,
    out_type=jax.ShapeDtypeStruct((R, J), jnp.float32),
    mesh=_mesh,
    compiler_params=pltpu.CompilerParams(needs_layout_passes=False),
    scratch_types=[
        pltpu.VMEM((J,), jnp.int32),
        pltpu.VMEM((J,), jnp.float32),
        pltpu.VMEM((J,), jnp.float32),
        pltpu.VMEM((L,), jnp.float32),
        pltpu.VMEM((J,), jnp.int32),
        pltpu.VMEM((J,), jnp.float32),
        pltpu.VMEM((J,), jnp.float32),
        pltpu.VMEM((J,), jnp.float32),
        pltpu.VMEM((BLK * K,), jnp.float32),
        pltpu.VMEM((BLK * K,), jnp.float32),
        pltpu.VMEM((BLK, J), jnp.float32),
        pltpu.VMEM((BLK, J), jnp.float32),
        pltpu.SemaphoreType.DMA,
        pltpu.SemaphoreType.DMA,
        pltpu.SemaphoreType.DMA,
        pltpu.SemaphoreType.DMA,
    ],
)
def _regrid(y_hbm, inds_hbm, w_hbm, mask_hbm, fill_hbm, out_hbm,
            s_inds, s_w, s_mask, s_fill, s_i0, s_a0, s_a1, s_c,
            y_v0, y_v1, o_v0, o_v1, isem0, isem1, osem0, osem1):
    wid = lax.axis_index("s") * NC + lax.axis_index("c")
    row0 = wid * RPW

    pltpu.sync_copy(inds_hbm, s_inds)
    pltpu.sync_copy(w_hbm, s_w)
    pltpu.sync_copy(mask_hbm, s_mask)
    pltpu.sync_copy(fill_hbm, s_fill)

    fillv = s_fill[...]
    for c in range(NCH):
        sl = pl.ds(c * L, L)
        iv = s_inds[sl]
        wv = s_w[sl]
        mv = s_mask[sl]
        a1 = wv * mv
        s_i0[sl] = iv - 1
        s_a0[sl] = mv - a1
        s_a1[sl] = a1
        s_c[sl] = fillv - fillv * mv

    isems = (isem0, isem1)
    osems = (osem0, osem1)
    y_bufs = (y_v0, y_v1)
    o_bufs = (o_v0, o_v1)

    def start_in(b):
        base = (row0 + b * BLK) * K
        return pltpu.async_copy(
            y_hbm.at[pl.ds(base, BLK * K)], y_bufs[b % 2], isems[b % 2])

    def start_out(b):
        return pltpu.async_copy(
            o_bufs[b % 2], out_hbm.at[pl.ds(row0 + b * BLK, BLK), :],
            osems[b % 2])

    in_d = start_in(0)
    out_d = [None, None]
    for b in range(NBLK):
        p = b % 2
        nxt = start_in(b + 1) if b + 1 < NBLK else None
        in_d.wait()
        if out_d[p] is not None:
            out_d[p].wait()
        yvp = y_bufs[p]
        ovp = o_bufs[p]

        def chunk_body(c, _, yvp=yvp, ovp=ovp):
            sl = pl.ds(c * L, L)
            i0 = s_i0[sl]
            a0 = s_a0[sl]
            a1 = s_a1[sl]
            cv = s_c[sl]

            @plsc.parallel_loop(0, BLK, unroll=8)
            def row_loop(r):
                bidx = i0 + r * K
                y0 = plsc.load_gather(yvp, [bidx])
                y1 = plsc.load_gather(yvp, [bidx + 1])
                ovp[r, sl] = cv + a0 * y0 + a1 * y1

            return 0

        lax.fori_loop(0, NCH, chunk_body, 0)
        out_d[p] = start_out(b)
        in_d = nxt
    out_d[0].wait()
    out_d[1].wait()


def kernel(y_data, inds, weights, mask, fill_value):
    y_flat = y_data.reshape(-1)
    inds32 = inds.astype(jnp.int32)
    w = weights.astype(jnp.float32)
    m = mask.astype(jnp.float32)
    fill16 = jnp.broadcast_to(jnp.asarray(fill_value, jnp.float32), (L,))
    return _regrid(y_flat, inds32, w, m, fill16)


# speedup vs baseline: 1.7554x; 1.1417x over previous
import functools

import jax
import jax.numpy as jnp
from jax import lax
from jax.experimental import pallas as pl
from jax.experimental.pallas import tpu as pltpu
from jax.experimental.pallas import tpu_sc as plsc

R = 32768
K = 64
J = 256
L = 16
NC = 2
NS = 16
NW = NC * NS
RPW = R // NW
BLK = 128
NBLK = RPW // BLK
NCH = J // L

_mesh = plsc.VectorSubcoreMesh(core_axis_name="c", subcore_axis_name="s")


@functools.partial(
    pl.kernel,
    out_type=jax.ShapeDtypeStruct((R, J), jnp.float32),
    mesh=_mesh,
    compiler_params=pltpu.CompilerParams(needs_layout_passes=False),
    scratch_types=[
        pltpu.VMEM((J,), jnp.int32),
        pltpu.VMEM((J,), jnp.float32),
        pltpu.VMEM((J,), jnp.float32),
        pltpu.VMEM((L,), jnp.float32),
        pltpu.VMEM((J,), jnp.int32),
        pltpu.VMEM((J,), jnp.float32),
        pltpu.VMEM((J,), jnp.float32),
        pltpu.VMEM((J,), jnp.float32),
        pltpu.VMEM((BLK, K), jnp.float32),
        pltpu.VMEM((BLK, K), jnp.float32),
        pltpu.VMEM((BLK, J), jnp.float32),
        pltpu.VMEM((BLK, J), jnp.float32),
        pltpu.SemaphoreType.DMA,
        pltpu.SemaphoreType.DMA,
        pltpu.SemaphoreType.DMA,
        pltpu.SemaphoreType.DMA,
    ],
)
def _regrid(y_hbm, inds_hbm, w_hbm, mask_hbm, fill_hbm, out_hbm,
            s_inds, s_w, s_mask, s_fill, s_i0, s_a0, s_a1, s_c,
            y_v0, y_v1, o_v0, o_v1, isem0, isem1, osem0, osem1):
    wid = lax.axis_index("s") * NC + lax.axis_index("c")
    row0 = wid * RPW

    pltpu.sync_copy(inds_hbm, s_inds)
    pltpu.sync_copy(w_hbm, s_w)
    pltpu.sync_copy(mask_hbm, s_mask)
    pltpu.sync_copy(fill_hbm, s_fill)

    fillv = s_fill[...]
    for c in range(NCH):
        sl = pl.ds(c * L, L)
        iv = s_inds[sl]
        wv = s_w[sl]
        mv = s_mask[sl]
        a1 = wv * mv
        s_i0[sl] = iv - 1
        s_a0[sl] = mv - a1
        s_a1[sl] = a1
        s_c[sl] = fillv - fillv * mv

    isems = (isem0, isem1)
    osems = (osem0, osem1)
    y_bufs = (y_v0, y_v1)
    o_bufs = (o_v0, o_v1)

    def start_in(b):
        return pltpu.async_copy(
            y_hbm.at[pl.ds(row0 + b * BLK, BLK), :], y_bufs[b % 2],
            isems[b % 2])

    def start_out(b):
        return pltpu.async_copy(
            o_bufs[b % 2], out_hbm.at[pl.ds(row0 + b * BLK, BLK), :],
            osems[b % 2])

    in_d = start_in(0)
    out_d = [None, None]
    for b in range(NBLK):
        p = b % 2
        nxt = start_in(b + 1) if b + 1 < NBLK else None
        in_d.wait()
        if out_d[p] is not None:
            out_d[p].wait()
        yvp = y_bufs[p]
        ovp = o_bufs[p]

        def chunk_body(c, _, yvp=yvp, ovp=ovp):
            sl = pl.ds(c * L, L)
            i0 = s_i0[sl]
            a0 = s_a0[sl]
            a1 = s_a1[sl]
            cv = s_c[sl]

            @plsc.parallel_loop(0, BLK, unroll=8)
            def row_loop(r):
                rv = jnp.full((L,), r, dtype=jnp.int32)
                y0 = plsc.load_gather(yvp, [rv, i0])
                y1 = plsc.load_gather(yvp, [rv, i0 + 1])
                ovp[r, sl] = cv + a0 * y0 + a1 * y1

            return 0

        lax.fori_loop(0, NCH, chunk_body, 0)
        out_d[p] = start_out(b)
        in_d = nxt
    out_d[0].wait()
    out_d[1].wait()


def kernel(y_data, inds, weights, mask, fill_value):
    inds32 = inds.astype(jnp.int32)
    w = weights.astype(jnp.float32)
    m = mask.astype(jnp.float32)
    fill16 = jnp.broadcast_to(jnp.asarray(fill_value, jnp.float32), (L,))
    return _regrid(y_data, inds32, w, m, fill16)
